# baseline (speedup 1.0000x reference)
import functools

import jax
import jax.numpy as jnp
from jax import lax
from jax.experimental import pallas as pl
from jax.experimental.pallas import tpu as pltpu
from jax.experimental.pallas import tpu_sc as plsc

N_NODES = 10000
N_EDGES = 160000
N_PAIRS = 170000
F_OUT = 128

CH = 10752
PPAD = 16 * CH
NJ = CH // 128
NV = CH // 16
HALF = 50_000_000
DUMMY = 100_000_000
TSIZE = 100_000_008
SROWS = 10240


def _sc_body(nodes_hbm, src_hbm, dst_hbm, wa_hbm,
             s32_hbm, ps_hbm, t_hbm,
             nodes_v, src_v, dst_v, key2_v, pval_v, e_v, t_v, s_local,
             wa_v, row_v, sem):
    c = lax.axis_index("c")
    s = lax.axis_index("s")
    base = s * CH

    pltpu.sync_copy(nodes_hbm, nodes_v)
    pltpu.sync_copy(wa_hbm, wa_v)
    pltpu.sync_copy(src_hbm.at[pl.ds(base, CH)], src_v)
    pltpu.sync_copy(dst_hbm.at[pl.ds(base, CH)], dst_v)

    def dot_body(i, carry):
        a1v, a2v = carry
        w = wa_v[pl.ds(i * 16, 16)]
        return (a1v + w * wa_v[pl.ds(128 + i * 16, 16)],
                a2v + w * wa_v[pl.ds(256 + i * 16, 16)])
    zero16 = jnp.zeros((16,), jnp.float32)
    acc1, acc2 = lax.fori_loop(0, 8, dot_body, (zero16, zero16))
    c1 = jnp.sum(acc1, axis=0)
    c2 = jnp.sum(acc2, axis=0)

    lanes = lax.iota(jnp.int32, 16)
    kdummy = DUMMY + c
    klo = c * HALF
    khi = klo + HALF

    def e_body(i, acc):
        sv = src_v[pl.ds(i * 16, 16)]
        dv = dst_v[pl.ds(i * 16, 16)]
        pv = base + i * 16 + lanes
        ns = plsc.load_gather(nodes_v, [sv])
        nd = plsc.load_gather(nodes_v, [dv])
        z = c1 * ns + c2 * nd
        z = jnp.maximum(z, z * jnp.float32(0.01))
        valid = pv < N_PAIRS
        e = jnp.where(valid, jnp.exp(z), jnp.float32(0.0))
        e_v[pl.ds(i * 16, 16)] = e
        key = sv * N_NODES + dv
        own = valid & (key >= klo) & (key < khi)
        key2_v[i // 8, pl.ds((i % 8) * 16, 16)] = jnp.where(own, key, kdummy)
        pval_v[pl.ds(i * 16, 16)] = pv
        return acc + e
    acc_e = lax.fori_loop(0, NV, e_body, zero16)

    row_v[pl.ds(0, 16)] = acc_e
    pltpu.sync_copy(row_v, ps_hbm.at[c * 16 + s])

    def scat_body(j, _):
        pltpu.async_copy(pval_v.at[pl.ds(j * 128, 128)],
                         t_hbm.at[key2_v.at[j]], sem).wait()
        return 0
    lax.fori_loop(0, NJ, scat_body, 0)

    plsc.subcore_barrier()

    def gath_body(j, _):
        pltpu.async_copy(t_hbm.at[key2_v.at[j]],
                         t_v.at[pl.ds(j * 128, 128)], sem).wait()
        return 0
    lax.fori_loop(0, NJ, gath_body, 0)

    def zero_body(k, _):
        s_local[pl.ds(k * 16, 16)] = zero16
        return 0
    lax.fori_loop(0, SROWS // 16, zero_body, 0)

    def acc_body(i, _):
        kv = key2_v[i // 8, pl.ds((i % 8) * 16, 16)]
        pv = pval_v[pl.ds(i * 16, 16)]
        tv = t_v[pl.ds(i * 16, 16)]
        m = (kv != kdummy) & (tv == pv)
        dv = dst_v[pl.ds(i * 16, 16)]
        sv = src_v[pl.ds(i * 16, 16)]
        w = e_v[pl.ds(i * 16, 16)] * plsc.load_gather(nodes_v, [dv])
        plsc.addupdate_scatter(s_local, [sv], w, mask=m)
        return 0
    lax.fori_loop(0, NV, acc_body, 0)

    pltpu.sync_copy(s_local, s32_hbm.at[c * 16 + s])


def _tc_body(s32_ref, ps_ref, w_ref, o_ref):
    denom = jnp.sum(ps_ref[:16, :])
    ssum = jnp.sum(s32_ref[...], axis=0, keepdims=True)
    ssum = ssum[:, :N_NODES] * (jnp.float32(1.0) / denom)
    out = lax.dot_general(ssum, w_ref[...], (((0,), (0,)), ((), ())),
                          preferred_element_type=jnp.float32)
    o_ref[0] = jnp.where(out > 0, out, out * jnp.float32(0.01))


@jax.jit
def kernel(x, src, dst, W_node, a):
    nodes = x[0, N_EDGES:]
    srcp = jnp.pad(src.astype(jnp.int32), (0, PPAD - N_PAIRS))
    dstp = jnp.pad(dst.astype(jnp.int32), (0, PPAD - N_PAIRS))
    wa = jnp.concatenate([W_node[0], a[:F_OUT, 0], a[F_OUT:, 0]])

    mesh = plsc.VectorSubcoreMesh(core_axis_name="c", subcore_axis_name="s",
                                  num_cores=2, num_subcores=16)
    sc = pl.kernel(
        _sc_body,
        mesh=mesh,
        compiler_params=pltpu.CompilerParams(needs_layout_passes=False),
        out_type=[
            jax.ShapeDtypeStruct((32, SROWS), jnp.float32),
            jax.ShapeDtypeStruct((32, 16), jnp.float32),
            jax.ShapeDtypeStruct((TSIZE,), jnp.int32),
        ],
        scratch_types=[
            pltpu.VMEM((N_NODES,), jnp.float32),
            pltpu.VMEM((CH,), jnp.int32),
            pltpu.VMEM((CH,), jnp.int32),
            pltpu.VMEM((NJ, 128), jnp.int32),
            pltpu.VMEM((CH,), jnp.int32),
            pltpu.VMEM((CH,), jnp.float32),
            pltpu.VMEM((CH,), jnp.int32),
            pltpu.VMEM((SROWS,), jnp.float32),
            pltpu.VMEM((384,), jnp.float32),
            pltpu.VMEM((16,), jnp.float32),
            pltpu.SemaphoreType.DMA,
        ],
    )
    s32, ps, _t = sc(nodes, srcp, dstp, wa)

    out = pl.pallas_call(
        _tc_body,
        out_shape=jax.ShapeDtypeStruct((1, N_NODES, F_OUT), jnp.float32),
    )(s32, ps, W_node)
    return out


# speedup vs baseline: 1.0000x; 1.0000x over previous
import functools

import jax
import jax.numpy as jnp
from jax import lax
from jax.experimental import pallas as pl
from jax.experimental.pallas import tpu as pltpu
from jax.experimental.pallas import tpu_sc as plsc

N_NODES = 10000
N_EDGES = 160000
N_PAIRS = 170000
F_OUT = 128

CH = 10752
PPAD = 16 * CH
NJ = CH // 128
NV = CH // 16
HALF = 50_000_000
DUMMY = 100_000_000
TSIZE = 100_000_008
SROWS = 10240


def _sc_body(nodes_hbm, src_hbm, dst_hbm, wa_hbm,
             s32_hbm, ps_hbm,
             t_hbm, nodes_v, src_v, dst_v, key2_v, pval_v, e_v, t_v, s_local,
             wa_v, row_v, sem):
    c = lax.axis_index("c")
    s = lax.axis_index("s")
    base = s * CH

    pltpu.sync_copy(nodes_hbm, nodes_v)
    pltpu.sync_copy(wa_hbm, wa_v)
    pltpu.sync_copy(src_hbm.at[pl.ds(base, CH)], src_v)
    pltpu.sync_copy(dst_hbm.at[pl.ds(base, CH)], dst_v)

    def dot_body(i, carry):
        a1v, a2v = carry
        w = wa_v[pl.ds(i * 16, 16)]
        return (a1v + w * wa_v[pl.ds(128 + i * 16, 16)],
                a2v + w * wa_v[pl.ds(256 + i * 16, 16)])
    zero16 = jnp.zeros((16,), jnp.float32)
    acc1, acc2 = lax.fori_loop(0, 8, dot_body, (zero16, zero16))
    c1 = jnp.sum(acc1, axis=0)
    c2 = jnp.sum(acc2, axis=0)

    lanes = lax.iota(jnp.int32, 16)
    kdummy = DUMMY + c
    klo = c * HALF
    khi = klo + HALF

    def e_body(i, acc):
        sv = src_v[pl.ds(i * 16, 16)]
        dv = dst_v[pl.ds(i * 16, 16)]
        pv = base + i * 16 + lanes
        ns = plsc.load_gather(nodes_v, [sv])
        nd = plsc.load_gather(nodes_v, [dv])
        z = c1 * ns + c2 * nd
        z = jnp.maximum(z, z * jnp.float32(0.01))
        valid = pv < N_PAIRS
        e = jnp.where(valid, jnp.exp(z), jnp.float32(0.0))
        e_v[pl.ds(i * 16, 16)] = e
        key = sv * N_NODES + dv
        own = valid & (key >= klo) & (key < khi)
        key2_v[i // 8, pl.ds((i % 8) * 16, 16)] = jnp.where(own, key, kdummy)
        pval_v[pl.ds(i * 16, 16)] = pv
        return acc + e
    acc_e = lax.fori_loop(0, NV, e_body, zero16)

    row_v[pl.ds(0, 16)] = acc_e
    pltpu.sync_copy(row_v, ps_hbm.at[c * 16 + s])

    def scat_body(j, _):
        pltpu.async_copy(pval_v.at[pl.ds(j * 128, 128)],
                         t_hbm.at[key2_v.at[j]], sem).wait()
        return 0
    lax.fori_loop(0, NJ, scat_body, 0)

    plsc.subcore_barrier()

    def gath_body(j, _):
        pltpu.async_copy(t_hbm.at[key2_v.at[j]],
                         t_v.at[pl.ds(j * 128, 128)], sem).wait()
        return 0
    lax.fori_loop(0, NJ, gath_body, 0)

    def zero_body(k, _):
        s_local[pl.ds(k * 16, 16)] = zero16
        return 0
    lax.fori_loop(0, SROWS // 16, zero_body, 0)

    def acc_body(i, _):
        kv = key2_v[i // 8, pl.ds((i % 8) * 16, 16)]
        pv = pval_v[pl.ds(i * 16, 16)]
        tv = t_v[pl.ds(i * 16, 16)]
        m = (kv != kdummy) & (tv == pv)
        dv = dst_v[pl.ds(i * 16, 16)]
        sv = src_v[pl.ds(i * 16, 16)]
        w = e_v[pl.ds(i * 16, 16)] * plsc.load_gather(nodes_v, [dv])
        plsc.addupdate_scatter(s_local, [sv], w, mask=m)
        return 0
    lax.fori_loop(0, NV, acc_body, 0)

    pltpu.sync_copy(s_local, s32_hbm.at[c * 16 + s])


def _tc_body(s32_ref, ps_ref, w_ref, o_ref):
    denom = jnp.sum(ps_ref[:16, :])
    ssum = jnp.sum(s32_ref[...], axis=0, keepdims=True)
    ssum = ssum[:, :N_NODES] * (jnp.float32(1.0) / denom)
    out = lax.dot_general(ssum, w_ref[...], (((0,), (0,)), ((), ())),
                          preferred_element_type=jnp.float32)
    o_ref[0] = jnp.where(out > 0, out, out * jnp.float32(0.01))


@jax.jit
def kernel(x, src, dst, W_node, a):
    nodes = x[0, N_EDGES:]
    srcp = jnp.pad(src.astype(jnp.int32), (0, PPAD - N_PAIRS))
    dstp = jnp.pad(dst.astype(jnp.int32), (0, PPAD - N_PAIRS))
    wa = jnp.concatenate([W_node[0], a[:F_OUT, 0], a[F_OUT:, 0]])

    mesh = plsc.VectorSubcoreMesh(core_axis_name="c", subcore_axis_name="s",
                                  num_cores=2, num_subcores=16)
    sc = pl.kernel(
        _sc_body,
        mesh=mesh,
        compiler_params=pltpu.CompilerParams(needs_layout_passes=False),
        out_type=[
            jax.ShapeDtypeStruct((32, SROWS), jnp.float32),
            jax.ShapeDtypeStruct((32, 16), jnp.float32),
        ],
        scratch_types=[
            pltpu.HBM((TSIZE,), jnp.int32),
            pltpu.VMEM((N_NODES,), jnp.float32),
            pltpu.VMEM((CH,), jnp.int32),
            pltpu.VMEM((CH,), jnp.int32),
            pltpu.VMEM((NJ, 128), jnp.int32),
            pltpu.VMEM((CH,), jnp.int32),
            pltpu.VMEM((CH,), jnp.float32),
            pltpu.VMEM((CH,), jnp.int32),
            pltpu.VMEM((SROWS,), jnp.float32),
            pltpu.VMEM((384,), jnp.float32),
            pltpu.VMEM((16,), jnp.float32),
            pltpu.SemaphoreType.DMA,
        ],
    )
    s32, ps = sc(nodes, srcp, dstp, wa)

    out = pl.pallas_call(
        _tc_body,
        out_shape=jax.ShapeDtypeStruct((1, N_NODES, F_OUT), jnp.float32),
    )(s32, ps, W_node)
    return out


# speedup vs baseline: 1.0003x; 1.0002x over previous
import functools

import jax
import jax.numpy as jnp
from jax import lax
from jax.experimental import pallas as pl
from jax.experimental.pallas import tpu as pltpu
from jax.experimental.pallas import tpu_sc as plsc

N_NODES = 10000
N_EDGES = 160000
N_PAIRS = 170000
F_OUT = 128

CH = 10752
PPAD = 16 * CH
NJ = CH // 128
NV = CH // 16
HALF = 50_000_000
DUMMY = 100_000_000
TSIZE = 100_000_008
SROWS = 10240


def _sc_body(nodes_hbm, src_hbm, dst_hbm, wa_hbm,
             s32_hbm, ps_hbm,
             t_hbm, nodes_v, src_v, dst_v, key2_v, pval_v, e_v, t_v, s_local,
             wa_v, row_v, sem):
    c = lax.axis_index("c")
    s = lax.axis_index("s")
    base = s * CH

    pltpu.sync_copy(nodes_hbm, nodes_v)
    pltpu.sync_copy(wa_hbm, wa_v)
    pltpu.sync_copy(src_hbm.at[pl.ds(base, CH)], src_v)
    pltpu.sync_copy(dst_hbm.at[pl.ds(base, CH)], dst_v)

    def dot_body(i, carry):
        a1v, a2v = carry
        w = wa_v[pl.ds(i * 16, 16)]
        return (a1v + w * wa_v[pl.ds(128 + i * 16, 16)],
                a2v + w * wa_v[pl.ds(256 + i * 16, 16)])
    zero16 = jnp.zeros((16,), jnp.float32)
    acc1, acc2 = lax.fori_loop(0, 8, dot_body, (zero16, zero16))
    c1 = jnp.sum(acc1, axis=0)
    c2 = jnp.sum(acc2, axis=0)

    lanes = lax.iota(jnp.int32, 16)
    kdummy = DUMMY + c
    klo = c * HALF
    khi = klo + HALF

    def e_body(i, acc):
        sv = src_v[pl.ds(i * 16, 16)]
        dv = dst_v[pl.ds(i * 16, 16)]
        pv = base + i * 16 + lanes
        ns = plsc.load_gather(nodes_v, [sv])
        nd = plsc.load_gather(nodes_v, [dv])
        z = c1 * ns + c2 * nd
        z = jnp.maximum(z, z * jnp.float32(0.01))
        valid = pv < N_PAIRS
        e = jnp.where(valid, jnp.exp(z), jnp.float32(0.0))
        e_v[pl.ds(i * 16, 16)] = e
        key = sv * N_NODES + dv
        own = valid & (key >= klo) & (key < khi)
        key2_v[pl.ds(i * 16, 16)] = jnp.where(own, key, kdummy)
        pval_v[pl.ds(i * 16, 16)] = pv
        return acc + e
    acc_e = lax.fori_loop(0, NV, e_body, zero16)

    row_v[pl.ds(0, 16)] = acc_e
    pltpu.sync_copy(row_v, ps_hbm.at[c * 16 + s])

    pltpu.async_copy(pval_v, t_hbm.at[key2_v], sem).wait()

    plsc.subcore_barrier()

    pltpu.async_copy(t_hbm.at[key2_v], t_v, sem).wait()

    def zero_body(k, _):
        s_local[pl.ds(k * 16, 16)] = zero16
        return 0
    lax.fori_loop(0, SROWS // 16, zero_body, 0)

    def acc_body(i, _):
        kv = key2_v[pl.ds(i * 16, 16)]
        pv = pval_v[pl.ds(i * 16, 16)]
        tv = t_v[pl.ds(i * 16, 16)]
        m = (kv != kdummy) & (tv == pv)
        dv = dst_v[pl.ds(i * 16, 16)]
        sv = src_v[pl.ds(i * 16, 16)]
        w = e_v[pl.ds(i * 16, 16)] * plsc.load_gather(nodes_v, [dv])
        plsc.addupdate_scatter(s_local, [sv], w, mask=m)
        return 0
    lax.fori_loop(0, NV, acc_body, 0)

    pltpu.sync_copy(s_local, s32_hbm.at[c * 16 + s])


def _tc_body(s32_ref, ps_ref, w_ref, o_ref):
    denom = jnp.sum(ps_ref[:16, :])
    ssum = jnp.sum(s32_ref[...], axis=0, keepdims=True)
    ssum = ssum[:, :N_NODES] * (jnp.float32(1.0) / denom)
    out = lax.dot_general(ssum, w_ref[...], (((0,), (0,)), ((), ())),
                          preferred_element_type=jnp.float32)
    o_ref[0] = jnp.where(out > 0, out, out * jnp.float32(0.01))


@jax.jit
def kernel(x, src, dst, W_node, a):
    nodes = x[0, N_EDGES:]
    srcp = jnp.pad(src.astype(jnp.int32), (0, PPAD - N_PAIRS))
    dstp = jnp.pad(dst.astype(jnp.int32), (0, PPAD - N_PAIRS))
    wa = jnp.concatenate([W_node[0], a[:F_OUT, 0], a[F_OUT:, 0]])

    mesh = plsc.VectorSubcoreMesh(core_axis_name="c", subcore_axis_name="s",
                                  num_cores=2, num_subcores=16)
    sc = pl.kernel(
        _sc_body,
        mesh=mesh,
        compiler_params=pltpu.CompilerParams(needs_layout_passes=False),
        out_type=[
            jax.ShapeDtypeStruct((32, SROWS), jnp.float32),
            jax.ShapeDtypeStruct((32, 16), jnp.float32),
        ],
        scratch_types=[
            pltpu.HBM((TSIZE,), jnp.int32),
            pltpu.VMEM((N_NODES,), jnp.float32),
            pltpu.VMEM((CH,), jnp.int32),
            pltpu.VMEM((CH,), jnp.int32),
            pltpu.VMEM((CH,), jnp.int32),
            pltpu.VMEM((CH,), jnp.int32),
            pltpu.VMEM((CH,), jnp.float32),
            pltpu.VMEM((CH,), jnp.int32),
            pltpu.VMEM((SROWS,), jnp.float32),
            pltpu.VMEM((384,), jnp.float32),
            pltpu.VMEM((16,), jnp.float32),
            pltpu.SemaphoreType.DMA,
        ],
    )
    s32, ps = sc(nodes, srcp, dstp, wa)

    out = pl.pallas_call(
        _tc_body,
        out_shape=jax.ShapeDtypeStruct((1, N_NODES, F_OUT), jnp.float32),
    )(s32, ps, W_node)
    return out
